# Initial kernel scaffold; baseline (speedup 1.0000x reference)
import jax
import jax.numpy as jnp
from jax.experimental import pallas as pl


def kernel(x, encoding_matrix):
    raise NotImplementedError("write your pallas kernel here")


# speedup vs baseline: 8.2703x; 8.2703x over previous
import jax
import jax.numpy as jnp
from jax.experimental import pallas as pl
from jax.experimental.pallas import tpu as pltpu

CLIP = 64
P_SHIFTS = 16
N_SEMS = 4


def _band_dma_kernel(table_ref, out_ref, e2_ref, sems, *, S, C, D, P, K):
    t0 = table_ref[0:1, :]
    tmid = table_ref[1 : 2 * C, :]
    tlast = table_ref[2 * C : 2 * C + 1, :]
    for p in range(P):
        e2_ref[p, 0 : p + S - C, :] = jnp.broadcast_to(t0, (p + S - C, D))
        e2_ref[p, p + S - C : p + S - 1 + C, :] = tmid
        e2_ref[p, p + S - 1 + C :, :] = jnp.broadcast_to(tlast, (2 * S - (p + S - 1 + C), D))

    n = S // P

    def make_copy(t):
        r = t * P
        return pltpu.make_async_copy(
            e2_ref.at[:, pl.ds(S - 1 - r, S), :],
            out_ref.at[pl.ds(r, P)],
            sems.at[t % K],
        )

    def body(t, _):
        @pl.when(t >= K)
        def _():
            make_copy(t - K).wait()
        make_copy(t).start()
        return 0

    jax.lax.fori_loop(0, n, body, 0)

    def drain(k, _):
        make_copy(n - K + k).wait()
        return 0

    jax.lax.fori_loop(0, K, drain, 0)


def _rel_pos_encoding(table, S, C, D, interpret=False):
    P, K = P_SHIFTS, N_SEMS
    return pl.pallas_call(
        lambda t, o, e, s: _band_dma_kernel(t, o, e, s, S=S, C=C, D=D, P=P, K=K),
        in_specs=[pl.BlockSpec(memory_space=pltpu.VMEM)],
        out_specs=pl.BlockSpec(memory_space=pltpu.MemorySpace.HBM),
        out_shape=jax.ShapeDtypeStruct((S, S, D), table.dtype),
        scratch_shapes=[
            pltpu.VMEM((P, 2 * S, D), table.dtype),
            pltpu.SemaphoreType.DMA((N_SEMS,)),
        ],
        interpret=interpret,
    )(table)


def kernel(x, encoding_matrix):
    S = x.shape[1]
    D = encoding_matrix.shape[1]
    return _rel_pos_encoding(encoding_matrix, S, CLIP, D)
